# baseline (speedup 1.0000x reference)
import functools

import jax
import jax.numpy as jnp
from jax import lax
from jax.experimental import pallas as pl
from jax.experimental.pallas import tpu as pltpu, tpu_sc as plsc

N = 10000
E = 320000
DF = 128
H = 64
L = 3

_NODE_BLK = 1000
_EDGE_BLK = 4000

_NC, _NS = 2, 16
_NW = _NC * _NS
_EPW = E // _NW
_CH = 80
_NCHUNK = _EPW // _CH

_NPW = 320
_CAPP = 1152
_PENDTOT = _NW * _CAPP
_GRP = 128
_CAPW = 80 * _GRP
_BLK = 1024
_NBLK = 9
_TAIL = _EPW - _NBLK * _BLK

_i16 = lambda: lax.iota(jnp.int32, 16)
_SC_PARAMS = pltpu.CompilerParams(needs_layout_passes=False)


def _take16(x, idx):
    return lax.gather(
        x, idx[:, None],
        dimension_numbers=lax.GatherDimensionNumbers(
            offset_dims=(), collapsed_slice_dims=(0,), start_index_map=(0,)),
        slice_sizes=(1,),
        mode=lax.GatherScatterMode.PROMISE_IN_BOUNDS)


def _sload(ref, i):
    base = pl.multiple_of((i >> 4) << 4, 16)
    v = ref[pl.ds(base, 16)]
    return _take16(v, lax.broadcast(i & 15, (16,)))[0]


def _sstore(ref, i, val):
    base = pl.multiple_of((i >> 4) << 4, 16)
    v = ref[pl.ds(base, 16)]
    ref[pl.ds(base, 16)] = jnp.where(_i16() == (i & 15), val, v)


def _g_body(c_hbm, dst_hbm, src_hbm, p_hbm,
            idxd, idxs, bufD, bufS, bufP, semD, semS):
    wid = lax.axis_index("s") * _NC + lax.axis_index("c")

    def chunk(j, carry):
        base = pl.multiple_of(wid * _EPW + j * _CH, _CH)
        base2 = pl.multiple_of(wid * (_EPW // 2) + j * (_CH // 2), _CH // 2)
        pltpu.sync_copy(dst_hbm.at[pl.ds(base, _CH)], idxd)
        pltpu.sync_copy(src_hbm.at[pl.ds(base, _CH)], idxs)
        cpD = pltpu.async_copy(c_hbm.at[idxd], bufD, semD)
        cpS = pltpu.async_copy(c_hbm.at[idxs], bufS, semS)
        cpD.wait()
        cpS.wait()

        def pair(i, c2):
            for q in range(4):
                lo = pl.ds(q * 16, 16)
                hi = pl.ds(64 + q * 16, 16)
                bufP[i, lo] = bufD[2 * i, lo] + bufS[2 * i, hi]
                bufP[i, hi] = bufD[2 * i + 1, lo] + bufS[2 * i + 1, hi]
            return c2

        lax.fori_loop(0, _CH // 2, pair, 0)
        pltpu.sync_copy(bufP, p_hbm.at[pl.ds(base2, _CH // 2)])
        return carry

    lax.fori_loop(0, _NCHUNK, chunk, 0)


def _gather_add(c, dst, src):
    mesh = plsc.VectorSubcoreMesh(core_axis_name="c", subcore_axis_name="s")
    return pl.kernel(
        _g_body,
        out_type=jax.ShapeDtypeStruct((E // 2, 2 * H), jnp.float32),
        mesh=mesh,
        scratch_types=[
            pltpu.VMEM((_CH,), jnp.int32),
            pltpu.VMEM((_CH,), jnp.int32),
            pltpu.VMEM((_CH, 2 * H), jnp.float32),
            pltpu.VMEM((_CH, 2 * H), jnp.float32),
            pltpu.VMEM((_CH // 2, 2 * H), jnp.float32),
            pltpu.SemaphoreType.DMA,
            pltpu.SemaphoreType.DMA,
        ],
    )(c, dst, src)


def _r_body(dst_hbm, lists_hbm, ngrp_hbm, dbuf, pend, npv, goff, cbuf):
    wid = lax.axis_index("s") * _NC + lax.axis_index("c")
    base0 = wid * _EPW
    zi = jnp.zeros((16,), jnp.int32)
    ones = jnp.full((16,), 1, jnp.int32)

    def initp(i, c):
        pend[pl.ds(i * 16, 16)] = jnp.full((16,), _NPW, jnp.int32)
        return c
    lax.fori_loop(0, _PENDTOT // 16, initp, 0)
    npv[pl.ds(0, 16)] = zi
    npv[pl.ds(16, 16)] = zi
    goff[pl.ds(0, 16)] = zi
    goff[pl.ds(16, 16)] = zi

    def sub(k, ebase):
        d = dbuf[pl.ds(k * 16, 16)]
        o = d // _NPW
        eid = ebase + k * 16 + _i16()
        packed = (eid << 9) | (d - o * _NPW)
        o_s, perm = plsc.sort_key_val(o, _i16())
        p_s = _take16(packed, perm)
        prev = _take16(o_s, jnp.maximum(_i16() - 1, 0))
        is_start = (_i16() == 0) | (o_s != prev)
        start = plsc.cummax(jnp.where(is_start, _i16(), 0))
        rank = _i16() - start
        npo = plsc.load_gather(npv, [o_s])
        addr = o_s * _CAPP + npo + rank
        plsc.store_scatter(pend, [addr], p_s)
        plsc.addupdate_scatter(npv, [o_s], ones)

    def sweep(o, c):
        np_o = _sload(npv, o)
        gf = np_o >> 7
        g0 = _sload(goff, o)

        def flush1(g, c2):
            po = pl.multiple_of(o * _CAPP + g * _GRP, _GRP)
            go = pl.multiple_of((g0 + g) * _GRP, _GRP)
            pltpu.sync_copy(pend.at[pl.ds(po, _GRP)],
                            lists_hbm.at[wid, o, pl.ds(go, _GRP)])
            return c2
        lax.fori_loop(0, gf, flush1, 0)

        @pl.when(gf > 0)
        def _():
            for t in range(8):
                rem = pend[pl.ds(o * _CAPP + gf * _GRP + t * 16, 16)]
                pend[pl.ds(o * _CAPP + t * 16, 16)] = rem
            _sstore(npv, o, np_o & (_GRP - 1))
            _sstore(goff, o, g0 + gf)
        return c

    def block(b, c):
        base = pl.multiple_of(base0 + b * _BLK, 8)
        pltpu.sync_copy(dst_hbm.at[pl.ds(base, _BLK)], dbuf)
        lax.fori_loop(0, _BLK // 16, lambda k, cc: (sub(k, base), cc)[1], 0)
        lax.fori_loop(0, _NW, sweep, 0)
        return c
    lax.fori_loop(0, _NBLK, block, 0)

    tbase = pl.multiple_of(base0 + _NBLK * _BLK, 8)
    pltpu.sync_copy(dst_hbm.at[pl.ds(tbase, _TAIL)], dbuf.at[pl.ds(0, _TAIL)])
    lax.fori_loop(0, _TAIL // 16, lambda k, cc: (sub(k, tbase), cc)[1], 0)

    for t in range(2):
        cnt = goff[pl.ds(t * 16, 16)] * _GRP + npv[pl.ds(t * 16, 16)]
        cbuf[pl.ds(t * 16, 16)] = cnt

    def fsweep(o, c):
        np_o = _sload(npv, o)
        gf = (np_o + _GRP - 1) >> 7
        g0 = _sload(goff, o)

        def flush1(g, c2):
            po = pl.multiple_of(o * _CAPP + g * _GRP, _GRP)
            go = pl.multiple_of((g0 + g) * _GRP, _GRP)
            pltpu.sync_copy(pend.at[pl.ds(po, _GRP)],
                            lists_hbm.at[wid, o, pl.ds(go, _GRP)])
            return c2
        lax.fori_loop(0, gf, flush1, 0)
        return c
    lax.fori_loop(0, _NW, fsweep, 0)
    pltpu.sync_copy(cbuf, ngrp_hbm.at[wid])


def _route(dst):
    mesh = plsc.VectorSubcoreMesh(core_axis_name="c", subcore_axis_name="s")
    return pl.kernel(
        _r_body,
        out_type=(jax.ShapeDtypeStruct((_NW, _NW, _CAPW), jnp.int32),
                  jax.ShapeDtypeStruct((_NW, _NW), jnp.int32)),
        mesh=mesh,
        compiler_params=_SC_PARAMS,
        scratch_types=[
            pltpu.VMEM((_BLK,), jnp.int32),
            pltpu.VMEM((_PENDTOT,), jnp.int32),
            pltpu.VMEM((_NW,), jnp.int32),
            pltpu.VMEM((_NW,), jnp.int32),
            pltpu.VMEM((_NW,), jnp.int32),
        ],
    )(dst)


def _s_body(z2_hbm, lists_hbm, ngrp_hbm, agg_hbm,
            cntbuf, pkbuf, prbuf, locbuf, hfbuf, bufZ, agg_loc, sem):
    wid = lax.axis_index("s") * _NC + lax.axis_index("c")
    zf = jnp.zeros((16,), jnp.float32)

    def initz(i, c):
        for q in range(4):
            agg_loc[i, pl.ds(q * 16, 16)] = zf
        return c
    lax.fori_loop(0, _NPW, initz, 0)

    pltpu.sync_copy(ngrp_hbm, cntbuf)

    def cell(sidx, c):
        cv = cntbuf[sidx, pl.ds(pl.multiple_of((wid >> 4) << 4, 16), 16)]
        cnt = _take16(cv, lax.broadcast(wid & 15, (16,)))[0]
        ng = (cnt + _GRP - 1) >> 7

        def grp(g, c2):
            go = pl.multiple_of(g * _GRP, _GRP)
            pltpu.sync_copy(lists_hbm.at[sidx, wid, pl.ds(go, _GRP)], pkbuf)
            for k in range(_GRP // 16):
                pk = pkbuf[pl.ds(k * 16, 16)]
                prbuf[pl.ds(k * 16, 16)] = pk >> 10
                locbuf[pl.ds(k * 16, 16)] = pk & 511
                hfbuf[pl.ds(k * 16, 16)] = ((pk >> 9) & 1) * H
            pltpu.async_copy(z2_hbm.at[prbuf], bufZ, sem).wait()
            nthis = jnp.minimum(cnt - g * _GRP, _GRP)

            def edge(j, c3):
                loc = _sload(locbuf, j)
                hf = _sload(hfbuf, j)
                for q in range(4):
                    a = agg_loc[loc, pl.ds(q * 16, 16)]
                    z = bufZ[j, pl.ds(hf + q * 16, 16)]
                    agg_loc[loc, pl.ds(q * 16, 16)] = jnp.maximum(a, z)
                return c3
            lax.fori_loop(0, nthis, edge, 0)
            return c2
        lax.fori_loop(0, ng, grp, 0)
        return c
    lax.fori_loop(0, _NW, cell, 0)

    obase = pl.multiple_of(wid * _NPW, _NPW)
    pltpu.sync_copy(agg_loc, agg_hbm.at[pl.ds(obase, _NPW)])


def _scatter_max(z2, lists, ngrp):
    mesh = plsc.VectorSubcoreMesh(core_axis_name="c", subcore_axis_name="s")
    return pl.kernel(
        _s_body,
        out_type=jax.ShapeDtypeStruct((_NW * _NPW, H), jnp.float32),
        mesh=mesh,
        compiler_params=_SC_PARAMS,
        scratch_types=[
            pltpu.VMEM((_NW, _NW), jnp.int32),
            pltpu.VMEM((_GRP,), jnp.int32),
            pltpu.VMEM((_GRP,), jnp.int32),
            pltpu.VMEM((_GRP,), jnp.int32),
            pltpu.VMEM((_GRP,), jnp.int32),
            pltpu.VMEM((_GRP, 2 * H), jnp.float32),
            pltpu.VMEM((_NPW, H), jnp.float32),
            pltpu.SemaphoreType.DMA,
        ],
    )(z2, lists, ngrp)


def _mm_kernel(x_ref, wt_ref, b_ref, o_ref, *, relu_in, relu_out):
    x = x_ref[...]
    if relu_in:
        x = jnp.maximum(x, 0.0)
    acc = jnp.dot(x, wt_ref[...], preferred_element_type=jnp.float32)
    acc = acc + b_ref[...]
    if relu_out:
        acc = jnp.maximum(acc, 0.0)
    o_ref[...] = acc


def _linear(x, wt, b, blk, relu_in=False, relu_out=False):
    m, k = x.shape
    n = wt.shape[1]
    return pl.pallas_call(
        functools.partial(_mm_kernel, relu_in=relu_in, relu_out=relu_out),
        grid=(m // blk,),
        in_specs=[
            pl.BlockSpec((blk, k), lambda i: (i, 0)),
            pl.BlockSpec((k, n), lambda i: (0, 0)),
            pl.BlockSpec((1, n), lambda i: (0, 0)),
        ],
        out_specs=pl.BlockSpec((blk, n), lambda i: (i, 0)),
        out_shape=jax.ShapeDtypeStruct((m, n), jnp.float32),
    )(x, wt, b.reshape(1, n))


def _init_kernel(x_ref, wpt_ref, bp_ref, wct_ref, bc_ref, h_ref, c_ref):
    h = jnp.maximum(
        jnp.dot(x_ref[...], wpt_ref[...], preferred_element_type=jnp.float32)
        + bp_ref[...], 0.0)
    h_ref[...] = h
    c_ref[...] = jnp.dot(h, wct_ref[...],
                         preferred_element_type=jnp.float32) + bc_ref[...]


def _init(x, wpt, bp, wct, bc):
    blk = _NODE_BLK
    return pl.pallas_call(
        _init_kernel,
        grid=(N // blk,),
        in_specs=[
            pl.BlockSpec((blk, DF), lambda i: (i, 0)),
            pl.BlockSpec((DF, H), lambda i: (0, 0)),
            pl.BlockSpec((1, H), lambda i: (0, 0)),
            pl.BlockSpec((H, 2 * H), lambda i: (0, 0)),
            pl.BlockSpec((1, 2 * H), lambda i: (0, 0)),
        ],
        out_specs=[
            pl.BlockSpec((blk, H), lambda i: (i, 0)),
            pl.BlockSpec((blk, 2 * H), lambda i: (i, 0)),
        ],
        out_shape=[
            jax.ShapeDtypeStruct((N, H), jnp.float32),
            jax.ShapeDtypeStruct((N, 2 * H), jnp.float32),
        ],
    )(x, wpt, bp.reshape(1, H), wct, bc.reshape(1, 2 * H))


def _head_kernel(agg_ref, h_ref, wct_ref, bc_ref, hn_ref, c_ref):
    hn = agg_ref[...] + h_ref[...]
    hn_ref[...] = hn
    c_ref[...] = jnp.dot(hn, wct_ref[...],
                         preferred_element_type=jnp.float32) + bc_ref[...]


def _head(agg, h, wct, bc):
    blk = _NODE_BLK
    return pl.pallas_call(
        _head_kernel,
        grid=(N // blk,),
        in_specs=[
            pl.BlockSpec((blk, H), lambda i: (i, 0)),
            pl.BlockSpec((blk, H), lambda i: (i, 0)),
            pl.BlockSpec((H, 2 * H), lambda i: (0, 0)),
            pl.BlockSpec((1, 2 * H), lambda i: (0, 0)),
        ],
        out_specs=[
            pl.BlockSpec((blk, H), lambda i: (i, 0)),
            pl.BlockSpec((blk, 2 * H), lambda i: (i, 0)),
        ],
        out_shape=[
            jax.ShapeDtypeStruct((N, H), jnp.float32),
            jax.ShapeDtypeStruct((N, 2 * H), jnp.float32),
        ],
    )(agg, h, wct, bc.reshape(1, 2 * H))


def _out_kernel(agg_ref, h_ref, w1_ref, b1_ref, w2_ref, b2_ref, o_ref):
    hn = agg_ref[...] + h_ref[...]
    t = jnp.maximum(
        jnp.dot(hn, w1_ref[...], preferred_element_type=jnp.float32)
        + b1_ref[...], 0.0)
    o_ref[...] = jnp.dot(t, w2_ref[...],
                         preferred_element_type=jnp.float32) + b2_ref[...]


def _out(agg, h, w1t, b1, w2t, b2):
    blk = _NODE_BLK
    hh = H // 2
    return pl.pallas_call(
        _out_kernel,
        grid=(N // blk,),
        in_specs=[
            pl.BlockSpec((blk, H), lambda i: (i, 0)),
            pl.BlockSpec((blk, H), lambda i: (i, 0)),
            pl.BlockSpec((H, hh), lambda i: (0, 0)),
            pl.BlockSpec((1, hh), lambda i: (0, 0)),
            pl.BlockSpec((hh, 1), lambda i: (0, 0)),
            pl.BlockSpec((1, 1), lambda i: (0, 0)),
        ],
        out_specs=pl.BlockSpec((blk, 1), lambda i: (i, 0)),
        out_shape=jax.ShapeDtypeStruct((N, 1), jnp.float32),
    )(agg, h, w1t, b1.reshape(1, hh), w2t, b2.reshape(1, 1))


def kernel(x, edge_index, batch, params):
    src = edge_index[0]
    dst = edge_index[1]

    lists, ngrp = _route(dst)

    def fold(p):
        s = p['g'] * jax.lax.rsqrt(p['rv'] + 1e-5)
        t = p['be'] - p['rm'] * s
        W1a = p['W1'][:, :H]
        W1b = p['W1'][:, H:]
        Wa = (W1a - W1b) * s[:, None]
        Wb = W1b * s[:, None]
        c = p['b1'] * s + t
        wct = jnp.concatenate([Wa.T, Wb.T], axis=1)
        bc = jnp.concatenate([c, jnp.zeros((H,), jnp.float32)])
        w2t = p['W2'].T
        z = jnp.zeros_like(w2t)
        w2blk = jnp.block([[w2t, z], [z, w2t]])
        b22 = jnp.concatenate([p['b2'], p['b2']])
        return wct, bc, w2blk, b22

    folds = [fold(p) for p in params['layers']]
    h, C = _init(x, params['Wp'].T, params['bp'], folds[0][0], folds[0][1])

    for l in range(L):
        _, _, w2b, b22 = folds[l]
        P = _gather_add(C, dst, src)
        Z2 = _linear(P, w2b, b22, _EDGE_BLK, relu_in=True)
        aggp = _scatter_max(Z2, lists, ngrp)
        agg = lax.slice(aggp, (0, 0), (N, H))

        if l + 1 < L:
            h, C = _head(agg, h, folds[l + 1][0], folds[l + 1][1])
        else:
            o = _out(agg, h, params['Wo1'].T, params['bo1'],
                     params['Wo2'].T, params['bo2'])
    return jnp.squeeze(o, -1)
